# Initial kernel scaffold; baseline (speedup 1.0000x reference)
import jax
import jax.numpy as jnp
from jax import lax
from jax.experimental import pallas as pl
from jax.experimental.pallas import tpu as pltpu
from jax.experimental.pallas import tpu_sc as plsc

N_NODES = 10000
N_EDGES = 320000
DIM = 128
NUM_REL = 8
NUM_BASES = 4

NUM_CORES = 2
NUM_SUBCORES = 16
NUM_WORKERS = NUM_CORES * NUM_SUBCORES

NPAD = 10240
ROWS_PER_SUB = NPAD // NUM_SUBCORES
EDGE_WIN = 128
E_PAD = 323584
WINDOWS = E_PAD // EDGE_WIN
WPW = WINDOWS // NUM_WORKERS
DEG_W = 16

ROW_BLK = 2000


def _xw_body(coeff_ref, x_ref, bases_ref, out_ref):
    r = pl.program_id(0)
    w = coeff_ref[r, 0] * bases_ref[0]
    for b in range(1, NUM_BASES):
        w = w + coeff_ref[r, b] * bases_ref[b]
    out_ref[...] = jnp.dot(x_ref[...], w, preferred_element_type=jnp.float32)


def _xw_table(x, bases, coefficients):
    nblk = N_NODES // ROW_BLK
    return pl.pallas_call(
        _xw_body,
        grid=(NUM_REL, nblk),
        in_specs=[
            pl.BlockSpec(memory_space=pltpu.SMEM),
            pl.BlockSpec((ROW_BLK, DIM), lambda r, i: (i, 0)),
            pl.BlockSpec((NUM_BASES, DIM, DIM), lambda r, i: (0, 0, 0)),
        ],
        out_specs=pl.BlockSpec((ROW_BLK, DIM), lambda r, i: (r * nblk + i, 0)),
        out_shape=jax.ShapeDtypeStruct((NUM_REL * N_NODES, DIM), jnp.float32),
    )(coefficients, x, bases)


def _sc_aggregate(xw, gidx2d, dst2d):
    mesh = plsc.VectorSubcoreMesh(core_axis_name="c", subcore_axis_name="s")

    @pl.kernel(
        out_type=(
            jax.ShapeDtypeStruct((NUM_CORES, NPAD, DIM), jnp.float32),
            jax.ShapeDtypeStruct((NUM_CORES, NPAD, DEG_W), jnp.float32),
        ),
        mesh=mesh,
        scratch_types=[
            pltpu.VMEM((WPW, EDGE_WIN), jnp.int32),
            pltpu.VMEM((WPW, EDGE_WIN), jnp.int32),
            pltpu.VMEM((EDGE_WIN, DIM), jnp.float32),
            pltpu.VMEM((EDGE_WIN, DEG_W), jnp.float32),
            pltpu.VMEM((EDGE_WIN, DIM), jnp.float32),
            pltpu.VMEM((ROWS_PER_SUB, DEG_W), jnp.float32),
            pltpu.VMEM_SHARED((NPAD, DIM), jnp.float32),
            pltpu.VMEM_SHARED((NPAD, DEG_W), jnp.float32),
        ],
    )
    def sc_kernel(xw_hbm, gi_hbm, di_hbm, acc_hbm, deg_hbm,
                  gi_v, di_v, gbuf, ones_v, zblk, zdeg, acc_sh, deg_sh):
        c = lax.axis_index("c")
        s = lax.axis_index("s")
        wid = c * NUM_SUBCORES + s
        base = s * ROWS_PER_SUB

        @pl.loop(0, EDGE_WIN)
        def _(i):
            @pl.loop(0, DIM // 16)
            def _(j):
                zblk[i, pl.ds(j * 16, 16)] = jnp.zeros((16,), jnp.float32)
            ones_v[i, :] = jnp.ones((DEG_W,), jnp.float32)

        @pl.loop(0, ROWS_PER_SUB)
        def _(i):
            zdeg[i, :] = jnp.zeros((DEG_W,), jnp.float32)

        for kk in range(ROWS_PER_SUB // EDGE_WIN):
            pltpu.sync_copy(zblk, acc_sh.at[pl.ds(base + kk * EDGE_WIN, EDGE_WIN)])
        pltpu.sync_copy(zdeg, deg_sh.at[pl.ds(base, ROWS_PER_SUB)])

        row0 = wid * WPW
        pltpu.sync_copy(gi_hbm.at[pl.ds(row0, WPW)], gi_v)
        pltpu.sync_copy(di_hbm.at[pl.ds(row0, WPW)], di_v)

        plsc.subcore_barrier()

        @pl.loop(0, WPW)
        def _(t):
            pltpu.sync_copy(xw_hbm.at[gi_v.at[t]], gbuf)
            pltpu.sync_copy(gbuf, acc_sh.at[di_v.at[t]], add=True)
            pltpu.sync_copy(ones_v, deg_sh.at[di_v.at[t]], add=True)

        plsc.subcore_barrier()

        pltpu.sync_copy(acc_sh.at[pl.ds(base, ROWS_PER_SUB)],
                        acc_hbm.at[c, pl.ds(base, ROWS_PER_SUB)])
        pltpu.sync_copy(deg_sh.at[pl.ds(base, ROWS_PER_SUB)],
                        deg_hbm.at[c, pl.ds(base, ROWS_PER_SUB)])

    return sc_kernel(xw, gidx2d, dst2d)


def _fin_body(acc_ref, deg_ref, x_ref, sl_ref, out_ref):
    a = acc_ref[0] + acc_ref[1]
    d = deg_ref[0] + deg_ref[1]
    dinv = 1.0 / jnp.maximum(d[:, :1], 1.0)
    out_ref[...] = a * dinv + jnp.dot(x_ref[...], sl_ref[...],
                                      preferred_element_type=jnp.float32)


def _finalize(acc, deg, x, self_loop):
    nblk = N_NODES // ROW_BLK
    return pl.pallas_call(
        _fin_body,
        grid=(nblk,),
        in_specs=[
            pl.BlockSpec((NUM_CORES, ROW_BLK, DIM), lambda i: (0, i, 0)),
            pl.BlockSpec((NUM_CORES, ROW_BLK, DEG_W), lambda i: (0, i, 0)),
            pl.BlockSpec((ROW_BLK, DIM), lambda i: (i, 0)),
            pl.BlockSpec((DIM, DIM), lambda i: (0, 0)),
        ],
        out_specs=pl.BlockSpec((ROW_BLK, DIM), lambda i: (i, 0)),
        out_shape=jax.ShapeDtypeStruct((N_NODES, DIM), jnp.float32),
    )(acc, deg, x, self_loop)


def kernel(x, edge_index, edge_type, bases, coefficients, self_loop):
    src = edge_index[0]
    dst = edge_index[1]
    gidx = edge_type.astype(jnp.int32) * N_NODES + src.astype(jnp.int32)
    pad = E_PAD - N_EDGES
    gidx2d = jnp.concatenate(
        [gidx, jnp.zeros((pad,), jnp.int32)]).reshape(WINDOWS, EDGE_WIN)
    dst2d = jnp.concatenate(
        [dst.astype(jnp.int32), jnp.full((pad,), N_NODES, jnp.int32)]
    ).reshape(WINDOWS, EDGE_WIN)

    xw = _xw_table(x, bases, coefficients)
    acc, deg = _sc_aggregate(xw, gidx2d, dst2d)
    return _finalize(acc, deg, x, self_loop)


# speedup vs baseline: 13.3392x; 13.3392x over previous
import jax
import jax.numpy as jnp
from jax import lax
from jax.experimental import pallas as pl
from jax.experimental.pallas import tpu as pltpu
from jax.experimental.pallas import tpu_sc as plsc

N_NODES = 10000
N_EDGES = 320000
DIM = 128
NUM_REL = 8
NUM_BASES = 4

NUM_CORES = 2
NUM_SUBCORES = 16
NUM_WORKERS = NUM_CORES * NUM_SUBCORES

NPAD = 10240
ROWS_PER_SUB = NPAD // NUM_SUBCORES
EDGE_WIN = 128
E_PAD = 327680
WINDOWS = E_PAD // EDGE_WIN
WPW = WINDOWS // NUM_WORKERS
DEG_W = 16

ROW_BLK = 2000


def _xw_body(coeff_ref, x_ref, bases_ref, out_ref):
    r = pl.program_id(0)
    w = coeff_ref[r, 0] * bases_ref[0]
    for b in range(1, NUM_BASES):
        w = w + coeff_ref[r, b] * bases_ref[b]
    out_ref[...] = jnp.dot(x_ref[...], w, preferred_element_type=jnp.float32)


def _xw_table(x, bases, coefficients):
    nblk = N_NODES // ROW_BLK
    return pl.pallas_call(
        _xw_body,
        grid=(NUM_REL, nblk),
        in_specs=[
            pl.BlockSpec(memory_space=pltpu.SMEM),
            pl.BlockSpec((ROW_BLK, DIM), lambda r, i: (i, 0)),
            pl.BlockSpec((NUM_BASES, DIM, DIM), lambda r, i: (0, 0, 0)),
        ],
        out_specs=pl.BlockSpec((ROW_BLK, DIM), lambda r, i: (r * nblk + i, 0)),
        out_shape=jax.ShapeDtypeStruct((NUM_REL * N_NODES, DIM), jnp.float32),
    )(coefficients, x, bases)


def _sc_aggregate(xw, gidx2d, dst2d):
    mesh = plsc.VectorSubcoreMesh(core_axis_name="c", subcore_axis_name="s")

    @pl.kernel(
        out_type=(
            jax.ShapeDtypeStruct((NUM_CORES, NPAD, DIM), jnp.float32),
            jax.ShapeDtypeStruct((NUM_CORES, NPAD, DEG_W), jnp.float32),
        ),
        mesh=mesh,
        compiler_params=pltpu.CompilerParams(use_tc_tiling_on_sc=False),
        scratch_types=[
            pltpu.VMEM((8, EDGE_WIN), jnp.int32),
            pltpu.VMEM((8, EDGE_WIN), jnp.int32),
            pltpu.VMEM((EDGE_WIN, DIM), jnp.float32),
            pltpu.VMEM((EDGE_WIN, DEG_W), jnp.float32),
            pltpu.VMEM((EDGE_WIN, DEG_W), jnp.float32),
            pltpu.VMEM_SHARED((NPAD, DIM), jnp.float32),
            pltpu.VMEM_SHARED((NPAD, DEG_W), jnp.float32),
        ],
    )
    def sc_kernel(xw_hbm, gi_hbm, di_hbm, acc_hbm, deg_hbm,
                  gi_v, di_v, gbuf, ones_v, zdeg, acc_sh, deg_sh):
        c = lax.axis_index("c")
        s = lax.axis_index("s")
        wid = c * NUM_SUBCORES + s
        base = s * ROWS_PER_SUB

        @pl.loop(0, EDGE_WIN)
        def _(i):
            @pl.loop(0, DIM // 16)
            def _(j):
                gbuf[i, pl.ds(j * 16, 16)] = jnp.zeros((16,), jnp.float32)
            ones_v[i, :] = jnp.ones((DEG_W,), jnp.float32)
            zdeg[i, :] = jnp.zeros((DEG_W,), jnp.float32)

        for kk in range(ROWS_PER_SUB // EDGE_WIN):
            pltpu.sync_copy(gbuf, acc_sh.at[pl.ds(base + kk * EDGE_WIN, EDGE_WIN)])
            pltpu.sync_copy(zdeg, deg_sh.at[pl.ds(base + kk * EDGE_WIN, EDGE_WIN)])

        row0 = wid * WPW

        plsc.subcore_barrier()

        @pl.loop(0, WPW // 8)
        def _(cb):
            pltpu.sync_copy(gi_hbm.at[pl.ds(row0 + cb * 8, 8)], gi_v)
            pltpu.sync_copy(di_hbm.at[pl.ds(row0 + cb * 8, 8)], di_v)

            @pl.loop(0, 8)
            def _(t):
                pltpu.sync_copy(xw_hbm.at[gi_v.at[t]], gbuf)
                pltpu.sync_copy(gbuf, acc_sh.at[di_v.at[t]], add=True)
                pltpu.sync_copy(ones_v, deg_sh.at[di_v.at[t]], add=True)

        plsc.subcore_barrier()

        pltpu.sync_copy(acc_sh.at[pl.ds(base, ROWS_PER_SUB)],
                        acc_hbm.at[c, pl.ds(base, ROWS_PER_SUB)])
        pltpu.sync_copy(deg_sh.at[pl.ds(base, ROWS_PER_SUB)],
                        deg_hbm.at[c, pl.ds(base, ROWS_PER_SUB)])

    return sc_kernel(xw, gidx2d, dst2d)


def _fin_body(acc_ref, deg_ref, x_ref, sl_ref, out_ref):
    a = acc_ref[0] + acc_ref[1]
    d = deg_ref[0] + deg_ref[1]
    dinv = 1.0 / jnp.maximum(d[:, :1], 1.0)
    out_ref[...] = a * dinv + jnp.dot(x_ref[...], sl_ref[...],
                                      preferred_element_type=jnp.float32)


def _finalize(acc, deg, x, self_loop):
    nblk = N_NODES // ROW_BLK
    return pl.pallas_call(
        _fin_body,
        grid=(nblk,),
        in_specs=[
            pl.BlockSpec((NUM_CORES, ROW_BLK, DIM), lambda i: (0, i, 0)),
            pl.BlockSpec((NUM_CORES, ROW_BLK, DEG_W), lambda i: (0, i, 0)),
            pl.BlockSpec((ROW_BLK, DIM), lambda i: (i, 0)),
            pl.BlockSpec((DIM, DIM), lambda i: (0, 0)),
        ],
        out_specs=pl.BlockSpec((ROW_BLK, DIM), lambda i: (i, 0)),
        out_shape=jax.ShapeDtypeStruct((N_NODES, DIM), jnp.float32),
    )(acc, deg, x, self_loop)


def kernel(x, edge_index, edge_type, bases, coefficients, self_loop):
    src = edge_index[0]
    dst = edge_index[1]
    gidx = edge_type.astype(jnp.int32) * N_NODES + src.astype(jnp.int32)
    pad = E_PAD - N_EDGES
    gidx2d = jnp.concatenate(
        [gidx, jnp.zeros((pad,), jnp.int32)]).reshape(WINDOWS, EDGE_WIN)
    dst2d = jnp.concatenate(
        [dst.astype(jnp.int32), jnp.full((pad,), N_NODES, jnp.int32)]
    ).reshape(WINDOWS, EDGE_WIN)

    xw = _xw_table(x, bases, coefficients)
    acc, deg = _sc_aggregate(xw, gidx2d, dst2d)
    return _finalize(acc, deg, x, self_loop)


# speedup vs baseline: 14.2741x; 1.0701x over previous
import jax
import jax.numpy as jnp
from jax import lax
from jax.experimental import pallas as pl
from jax.experimental.pallas import tpu as pltpu
from jax.experimental.pallas import tpu_sc as plsc

N_NODES = 10000
N_EDGES = 320000
DIM = 128
NUM_REL = 8
NUM_BASES = 4

NUM_CORES = 2
NUM_SUBCORES = 16
NUM_WORKERS = NUM_CORES * NUM_SUBCORES

NPAD = 10240
ROWS_PER_SUB = NPAD // NUM_SUBCORES
EDGE_WIN = 128
E_PAD = 327680
WINDOWS = E_PAD // EDGE_WIN
WPW = WINDOWS // NUM_WORKERS
DEG_W = 16

ROW_BLK = 2000


def _xw_body(coeff_ref, x_ref, bases_ref, out_ref):
    r = pl.program_id(0)
    w = coeff_ref[r, 0] * bases_ref[0]
    for b in range(1, NUM_BASES):
        w = w + coeff_ref[r, b] * bases_ref[b]
    out_ref[...] = jnp.dot(x_ref[...], w, preferred_element_type=jnp.float32)


def _xw_table(x, bases, coefficients):
    nblk = N_NODES // ROW_BLK
    return pl.pallas_call(
        _xw_body,
        grid=(NUM_REL, nblk),
        in_specs=[
            pl.BlockSpec(memory_space=pltpu.SMEM),
            pl.BlockSpec((ROW_BLK, DIM), lambda r, i: (i, 0)),
            pl.BlockSpec((NUM_BASES, DIM, DIM), lambda r, i: (0, 0, 0)),
        ],
        out_specs=pl.BlockSpec((ROW_BLK, DIM), lambda r, i: (r * nblk + i, 0)),
        out_shape=jax.ShapeDtypeStruct((NUM_REL * N_NODES, DIM), jnp.float32),
    )(coefficients, x, bases)


def _sc_aggregate(xw, gidx2d, dst2d, zacc, zdeg):
    mesh = plsc.VectorSubcoreMesh(core_axis_name="c", subcore_axis_name="s")

    @pl.kernel(
        out_type=(
            jax.ShapeDtypeStruct((NUM_CORES, NPAD, DIM), jnp.float32),
            jax.ShapeDtypeStruct((NUM_CORES, NPAD, DEG_W), jnp.float32),
        ),
        mesh=mesh,
        compiler_params=pltpu.CompilerParams(use_tc_tiling_on_sc=False),
        scratch_types=[
            pltpu.VMEM((8, EDGE_WIN), jnp.int32),
            pltpu.VMEM((8, EDGE_WIN), jnp.int32),
            pltpu.VMEM((EDGE_WIN, DIM), jnp.float32),
            pltpu.VMEM((EDGE_WIN, DIM), jnp.float32),
            pltpu.VMEM((EDGE_WIN, DEG_W), jnp.float32),
            pltpu.VMEM_SHARED((NPAD, DIM), jnp.float32),
            pltpu.VMEM_SHARED((NPAD, DEG_W), jnp.float32),
            pltpu.SemaphoreType.DMA,
            pltpu.SemaphoreType.DMA,
        ],
    )
    def sc_kernel(xw_hbm, gi_hbm, di_hbm, zacc_hbm, zdeg_hbm, acc_hbm, deg_hbm,
                  gi_v, di_v, gbuf0, gbuf1, ones_v, acc_sh, deg_sh,
                  gsem0, gsem1):
        c = lax.axis_index("c")
        s = lax.axis_index("s")
        wid = c * NUM_SUBCORES + s
        base = s * ROWS_PER_SUB
        gbufs = (gbuf0, gbuf1)
        gsems = (gsem0, gsem1)

        @pl.loop(0, EDGE_WIN)
        def _(i):
            ones_v[i, :] = jnp.ones((DEG_W,), jnp.float32)

        pltpu.sync_copy(zacc_hbm.at[pl.ds(base, ROWS_PER_SUB)],
                        acc_sh.at[pl.ds(base, ROWS_PER_SUB)])
        pltpu.sync_copy(zdeg_hbm.at[pl.ds(base, ROWS_PER_SUB)],
                        deg_sh.at[pl.ds(base, ROWS_PER_SUB)])

        row0 = wid * WPW

        plsc.subcore_barrier()

        @pl.loop(0, WPW // 8)
        def _(cb):
            pltpu.sync_copy(gi_hbm.at[pl.ds(row0 + cb * 8, 8)], gi_v)
            pltpu.sync_copy(di_hbm.at[pl.ds(row0 + cb * 8, 8)], di_v)

            copies = [None, None]
            copies[0] = pltpu.async_copy(xw_hbm.at[gi_v.at[0]], gbuf0, gsem0)
            for t in range(8):
                b = t & 1
                copies[b].wait()
                if t < 7:
                    copies[1 - b] = pltpu.async_copy(
                        xw_hbm.at[gi_v.at[t + 1]], gbufs[1 - b], gsems[1 - b])
                pltpu.sync_copy(gbufs[b], acc_sh.at[di_v.at[t]], add=True)
                pltpu.sync_copy(ones_v, deg_sh.at[di_v.at[t]], add=True)

        plsc.subcore_barrier()

        pltpu.sync_copy(acc_sh.at[pl.ds(base, ROWS_PER_SUB)],
                        acc_hbm.at[c, pl.ds(base, ROWS_PER_SUB)])
        pltpu.sync_copy(deg_sh.at[pl.ds(base, ROWS_PER_SUB)],
                        deg_hbm.at[c, pl.ds(base, ROWS_PER_SUB)])

    return sc_kernel(xw, gidx2d, dst2d, zacc, zdeg)


def _fin_body(acc_ref, deg_ref, x_ref, sl_ref, out_ref):
    a = acc_ref[0] + acc_ref[1]
    d = deg_ref[0] + deg_ref[1]
    dinv = 1.0 / jnp.maximum(d[:, :1], 1.0)
    out_ref[...] = a * dinv + jnp.dot(x_ref[...], sl_ref[...],
                                      preferred_element_type=jnp.float32)


def _finalize(acc, deg, x, self_loop):
    nblk = N_NODES // ROW_BLK
    return pl.pallas_call(
        _fin_body,
        grid=(nblk,),
        in_specs=[
            pl.BlockSpec((NUM_CORES, ROW_BLK, DIM), lambda i: (0, i, 0)),
            pl.BlockSpec((NUM_CORES, ROW_BLK, DEG_W), lambda i: (0, i, 0)),
            pl.BlockSpec((ROW_BLK, DIM), lambda i: (i, 0)),
            pl.BlockSpec((DIM, DIM), lambda i: (0, 0)),
        ],
        out_specs=pl.BlockSpec((ROW_BLK, DIM), lambda i: (i, 0)),
        out_shape=jax.ShapeDtypeStruct((N_NODES, DIM), jnp.float32),
    )(acc, deg, x, self_loop)


def kernel(x, edge_index, edge_type, bases, coefficients, self_loop):
    src = edge_index[0]
    dst = edge_index[1]
    gidx = edge_type.astype(jnp.int32) * N_NODES + src.astype(jnp.int32)
    pad = E_PAD - N_EDGES
    gidx2d = jnp.concatenate(
        [gidx, jnp.zeros((pad,), jnp.int32)]).reshape(WINDOWS, EDGE_WIN)
    dst2d = jnp.concatenate(
        [dst.astype(jnp.int32), jnp.full((pad,), N_NODES, jnp.int32)]
    ).reshape(WINDOWS, EDGE_WIN)

    xw = _xw_table(x, bases, coefficients)
    zacc = jnp.zeros((NPAD, DIM), jnp.float32)
    zdeg = jnp.zeros((NPAD, DEG_W), jnp.float32)
    acc, deg = _sc_aggregate(xw, gidx2d, dst2d, zacc, zdeg)
    return _finalize(acc, deg, x, self_loop)


# speedup vs baseline: 14.3052x; 1.0022x over previous
import jax
import jax.numpy as jnp
from jax import lax
from jax.experimental import pallas as pl
from jax.experimental.pallas import tpu as pltpu
from jax.experimental.pallas import tpu_sc as plsc

N_NODES = 10000
N_EDGES = 320000
DIM = 128
NUM_REL = 8
NUM_BASES = 4

NUM_CORES = 2
NUM_SUBCORES = 16
NUM_WORKERS = NUM_CORES * NUM_SUBCORES

NPAD = 10240
ROWS_PER_SUB = NPAD // NUM_SUBCORES
EDGE_WIN = 128
E_PAD = 327680
WINDOWS = E_PAD // EDGE_WIN
WPW = WINDOWS // NUM_WORKERS
DEG_W = 16

ROW_BLK = 2000


def _xw_body(coeff_ref, x_ref, bases_ref, out_ref):
    r = pl.program_id(0)
    w = coeff_ref[r, 0] * bases_ref[0]
    for b in range(1, NUM_BASES):
        w = w + coeff_ref[r, b] * bases_ref[b]
    out_ref[...] = jnp.dot(x_ref[...], w, preferred_element_type=jnp.float32)


def _xw_table(x, bases, coefficients):
    nblk = N_NODES // ROW_BLK
    return pl.pallas_call(
        _xw_body,
        grid=(NUM_REL, nblk),
        in_specs=[
            pl.BlockSpec(memory_space=pltpu.SMEM),
            pl.BlockSpec((ROW_BLK, DIM), lambda r, i: (i, 0)),
            pl.BlockSpec((NUM_BASES, DIM, DIM), lambda r, i: (0, 0, 0)),
        ],
        out_specs=pl.BlockSpec((ROW_BLK, DIM), lambda r, i: (r * nblk + i, 0)),
        out_shape=jax.ShapeDtypeStruct((NUM_REL * N_NODES, DIM), jnp.float32),
    )(coefficients, x, bases)


def _sc_aggregate(xw, gidx2d, dst2d, zacc, zdeg):
    mesh = plsc.VectorSubcoreMesh(core_axis_name="c", subcore_axis_name="s")

    @pl.kernel(
        out_type=(
            jax.ShapeDtypeStruct((NUM_CORES, NPAD, DIM), jnp.float32),
            jax.ShapeDtypeStruct((NUM_CORES, NPAD, DEG_W), jnp.float32),
        ),
        mesh=mesh,
        compiler_params=pltpu.CompilerParams(use_tc_tiling_on_sc=False),
        scratch_types=[
            pltpu.VMEM((8, EDGE_WIN), jnp.int32),
            pltpu.VMEM((8, EDGE_WIN), jnp.int32),
            pltpu.VMEM((EDGE_WIN, DIM), jnp.float32),
            pltpu.VMEM((EDGE_WIN, DIM), jnp.float32),
            pltpu.VMEM((EDGE_WIN, DEG_W), jnp.float32),
            pltpu.VMEM_SHARED((NPAD, DIM), jnp.float32),
            pltpu.VMEM_SHARED((NPAD, DEG_W), jnp.float32),
            pltpu.SemaphoreType.DMA,
            pltpu.SemaphoreType.DMA,
        ],
    )
    def sc_kernel(xw_hbm, gi_hbm, di_hbm, zacc_hbm, zdeg_hbm, acc_hbm, deg_hbm,
                  gi_v, di_v, gbuf0, gbuf1, ones_v, acc_sh, deg_sh,
                  gsem0, gsem1):
        c = lax.axis_index("c")
        s = lax.axis_index("s")
        wid = c * NUM_SUBCORES + s
        base = s * ROWS_PER_SUB
        gbufs = (gbuf0, gbuf1)
        gsems = (gsem0, gsem1)

        @pl.loop(0, EDGE_WIN)
        def _(i):
            ones_v[i, :] = jnp.ones((DEG_W,), jnp.float32)

        pltpu.sync_copy(zacc_hbm.at[pl.ds(base, ROWS_PER_SUB)],
                        acc_sh.at[pl.ds(base, ROWS_PER_SUB)])
        pltpu.sync_copy(zdeg_hbm.at[pl.ds(base, ROWS_PER_SUB)],
                        deg_sh.at[pl.ds(base, ROWS_PER_SUB)])

        row0 = wid * WPW

        plsc.subcore_barrier()

        @pl.loop(0, WPW // 8)
        def _(cb):
            pltpu.sync_copy(gi_hbm.at[pl.ds(row0 + cb * 8, 8)], gi_v)
            pltpu.sync_copy(di_hbm.at[pl.ds(row0 + cb * 8, 8)], di_v)

            copies = [None, None, None, None]

            def _issue(t):
                b = t & 1
                copies[2 * b] = pltpu.async_copy(
                    xw_hbm.at[gi_v.at[t, pl.ds(0, 64)]],
                    gbufs[b].at[pl.ds(0, 64)], gsems[b])
                copies[2 * b + 1] = pltpu.async_copy(
                    xw_hbm.at[gi_v.at[t, pl.ds(64, 64)]],
                    gbufs[b].at[pl.ds(64, 64)], gsems[b])

            _issue(0)
            for t in range(8):
                b = t & 1
                copies[2 * b].wait()
                copies[2 * b + 1].wait()
                if t < 7:
                    _issue(t + 1)
                pltpu.sync_copy(gbufs[b], acc_sh.at[di_v.at[t]], add=True)
                pltpu.sync_copy(ones_v, deg_sh.at[di_v.at[t]], add=True)

        plsc.subcore_barrier()

        pltpu.sync_copy(acc_sh.at[pl.ds(base, ROWS_PER_SUB)],
                        acc_hbm.at[c, pl.ds(base, ROWS_PER_SUB)])
        pltpu.sync_copy(deg_sh.at[pl.ds(base, ROWS_PER_SUB)],
                        deg_hbm.at[c, pl.ds(base, ROWS_PER_SUB)])

    return sc_kernel(xw, gidx2d, dst2d, zacc, zdeg)


def _fin_body(acc_ref, deg_ref, x_ref, sl_ref, out_ref):
    a = acc_ref[0] + acc_ref[1]
    d = deg_ref[0] + deg_ref[1]
    dinv = 1.0 / jnp.maximum(d[:, :1], 1.0)
    out_ref[...] = a * dinv + jnp.dot(x_ref[...], sl_ref[...],
                                      preferred_element_type=jnp.float32)


def _finalize(acc, deg, x, self_loop):
    nblk = N_NODES // ROW_BLK
    return pl.pallas_call(
        _fin_body,
        grid=(nblk,),
        in_specs=[
            pl.BlockSpec((NUM_CORES, ROW_BLK, DIM), lambda i: (0, i, 0)),
            pl.BlockSpec((NUM_CORES, ROW_BLK, DEG_W), lambda i: (0, i, 0)),
            pl.BlockSpec((ROW_BLK, DIM), lambda i: (i, 0)),
            pl.BlockSpec((DIM, DIM), lambda i: (0, 0)),
        ],
        out_specs=pl.BlockSpec((ROW_BLK, DIM), lambda i: (i, 0)),
        out_shape=jax.ShapeDtypeStruct((N_NODES, DIM), jnp.float32),
    )(acc, deg, x, self_loop)


def kernel(x, edge_index, edge_type, bases, coefficients, self_loop):
    src = edge_index[0]
    dst = edge_index[1]
    gidx = edge_type.astype(jnp.int32) * N_NODES + src.astype(jnp.int32)
    pad = E_PAD - N_EDGES
    gidx2d = jnp.concatenate(
        [gidx, jnp.zeros((pad,), jnp.int32)]).reshape(WINDOWS, EDGE_WIN)
    dst2d = jnp.concatenate(
        [dst.astype(jnp.int32), jnp.full((pad,), N_NODES, jnp.int32)]
    ).reshape(WINDOWS, EDGE_WIN)

    xw = _xw_table(x, bases, coefficients)
    zacc = jnp.zeros((NPAD, DIM), jnp.float32)
    zdeg = jnp.zeros((NPAD, DEG_W), jnp.float32)
    acc, deg = _sc_aggregate(xw, gidx2d, dst2d, zacc, zdeg)
    return _finalize(acc, deg, x, self_loop)
